# Initial kernel scaffold; baseline (speedup 1.0000x reference)
import functools

import jax
import jax.numpy as jnp
from jax import lax
from jax.experimental import pallas as pl
from jax.experimental.pallas import tpu as pltpu
from jax.experimental.pallas import tpu_sc as plsc

N_PAD = 10240
H = 64
NUM_SEG = 64
CHUNK = 128
NUM_WORKERS = 32
ROWS_PER_SUB = N_PAD // 16


def _sc_aggregate(y, src, dst, zeros):
    E = src.shape[0]
    nchunk = E // CHUNK
    iters = (nchunk + NUM_WORKERS - 1) // NUM_WORKERS
    mesh = plsc.VectorSubcoreMesh(core_axis_name="c", subcore_axis_name="s")

    @functools.partial(
        pl.kernel,
        mesh=mesh,
        out_type=jax.ShapeDtypeStruct((2, N_PAD, H), jnp.float32),
        scratch_types=[
            pltpu.VMEM((CHUNK,), jnp.int32),
            pltpu.VMEM((CHUNK,), jnp.int32),
            pltpu.VMEM((CHUNK, H), jnp.float32),
            pltpu.VMEM_SHARED((N_PAD, H), jnp.float32),
            pltpu.SemaphoreType.DMA,
        ],
    )
    def agg(y_hbm, src_hbm, dst_hbm, z_hbm, out_hbm, src_v, dst_v, rows_v,
            acc_sh, sem):
        c = lax.axis_index("c")
        s = lax.axis_index("s")
        wid = s * 2 + c
        pltpu.sync_copy(z_hbm.at[pl.ds(s * ROWS_PER_SUB, ROWS_PER_SUB)],
                        acc_sh.at[pl.ds(s * ROWS_PER_SUB, ROWS_PER_SUB)])
        plsc.subcore_barrier()

        def body(j, carry):
            k = wid + j * NUM_WORKERS

            @pl.when(k < nchunk)
            def _():
                base = k * CHUNK
                pltpu.sync_copy(src_hbm.at[pl.ds(base, CHUNK)], src_v)
                pltpu.sync_copy(dst_hbm.at[pl.ds(base, CHUNK)], dst_v)
                pltpu.async_copy(y_hbm.at[src_v], rows_v, sem).wait()
                pltpu.sync_copy(rows_v, acc_sh.at[dst_v], add=True)

            return carry

        lax.fori_loop(0, iters, body, 0)
        plsc.subcore_barrier()
        pltpu.sync_copy(acc_sh.at[pl.ds(s * ROWS_PER_SUB, ROWS_PER_SUB)],
                        out_hbm.at[c, pl.ds(s * ROWS_PER_SUB, ROWS_PER_SUB)])

    return agg(y, src, dst, zeros)


def _tc_matmul(x, w):
    BR = 1024
    K = x.shape[1]

    def body(x_ref, w_ref, o_ref):
        o_ref[...] = jnp.dot(x_ref[...], w_ref[...],
                             preferred_element_type=jnp.float32)

    return pl.pallas_call(
        body,
        grid=(N_PAD // BR,),
        in_specs=[
            pl.BlockSpec((BR, K), lambda i: (i, 0)),
            pl.BlockSpec((K, H), lambda i: (0, 0)),
        ],
        out_specs=pl.BlockSpec((BR, H), lambda i: (i, 0)),
        out_shape=jax.ShapeDtypeStruct((N_PAD, H), jnp.float32),
    )(x, w)


def _tc_mlp(y, a0, a1, b1, w2, b2, w1n):
    BR = 1024

    def body(y_ref, a0_ref, a1_ref, b1_ref, w2_ref, b2_ref, w1n_ref, o_ref):
        t = jnp.maximum(y_ref[...] + a0_ref[...] + a1_ref[...]
                        + b1_ref[...], 0.0)
        h = jnp.maximum(jnp.dot(t, w2_ref[...],
                                preferred_element_type=jnp.float32)
                        + b2_ref[...], 0.0)
        o_ref[...] = jnp.dot(h, w1n_ref[...],
                             preferred_element_type=jnp.float32)

    row = pl.BlockSpec((BR, H), lambda i: (i, 0))
    full = pl.BlockSpec((H, H), lambda i: (0, 0))
    vec = pl.BlockSpec((1, H), lambda i: (0, 0))
    return pl.pallas_call(
        body,
        grid=(N_PAD // BR,),
        in_specs=[row, row, row, vec, full, vec, full],
        out_specs=row,
        out_shape=jax.ShapeDtypeStruct((N_PAD, H), jnp.float32),
    )(y, a0, a1, b1.reshape(1, H), w2, b2.reshape(1, H), w1n)


def _tc_final(y, a0, a1, b1, w2, b2, batch2d, wa, ba, wp, bp, wc1, bc1,
              wc2, bc2, n_valid, out_dim):
    def body(y_ref, a0_ref, a1_ref, b1_ref, w2_ref, b2_ref, batch_ref,
             wa_ref, ba_ref, wp_ref, bp_ref, wc1_ref, bc1_ref, wc2_ref,
             bc2_ref, logits_ref, ge_ref, h_ref, scores_ref):
        t = jnp.maximum(y_ref[...] + a0_ref[...] + a1_ref[...]
                        + b1_ref[...], 0.0)
        h = jnp.maximum(jnp.dot(t, w2_ref[...],
                                preferred_element_type=jnp.float32)
                        + b2_ref[...], 0.0)
        h_ref[...] = h

        valid = (lax.broadcasted_iota(jnp.int32, (N_PAD, 1), 0) < n_valid)
        seg_ids = lax.broadcasted_iota(jnp.int32, (1, NUM_SEG), 1)
        onehot = jnp.logical_and(batch_ref[...] == seg_ids, valid)
        onehot_f = onehot.astype(jnp.float32)
        neg_inf = jnp.float32(-jnp.inf)

        def seg_softmax(logits):
            lg = jnp.where(valid, logits, 0.0)
            m = jnp.max(jnp.where(onehot, lg, neg_inf), axis=0,
                        keepdims=True)
            m = jnp.where(jnp.isfinite(m), m, 0.0)
            m_node = jnp.sum(onehot_f * m, axis=1, keepdims=True)
            e = jnp.exp(lg - m_node)
            e = jnp.where(valid, e, 0.0)
            d = jnp.sum(onehot_f * e, axis=0, keepdims=True)
            d_node = jnp.sum(onehot_f * d, axis=1, keepdims=True)
            return e / (d_node + 1e-16)

        sa = jnp.dot(h, wa_ref[...], preferred_element_type=jnp.float32) \
            + ba_ref[...]
        sp = jnp.dot(h, wp_ref[...], preferred_element_type=jnp.float32) \
            + bp_ref[...]
        scores_ref[...] = seg_softmax(sa)
        attn = seg_softmax(sp)

        ge = lax.dot_general(onehot_f, attn * h, (((0,), (0,)), ((), ())),
                             preferred_element_type=jnp.float32)
        ge_ref[...] = ge
        hc = jnp.maximum(jnp.dot(ge, wc1_ref[...],
                                 preferred_element_type=jnp.float32)
                         + bc1_ref[...], 0.0)
        logits_ref[...] = jnp.dot(hc, wc2_ref[...],
                                  preferred_element_type=jnp.float32) \
            + bc2_ref[...]

    return pl.pallas_call(
        body,
        out_shape=(
            jax.ShapeDtypeStruct((NUM_SEG, out_dim), jnp.float32),
            jax.ShapeDtypeStruct((NUM_SEG, H), jnp.float32),
            jax.ShapeDtypeStruct((N_PAD, H), jnp.float32),
            jax.ShapeDtypeStruct((N_PAD, 1), jnp.float32),
        ),
    )(y, a0, a1, b1.reshape(1, H), w2, b2.reshape(1, H), batch2d,
      wa, ba.reshape(1, 1), wp, bp.reshape(1, 1), wc1, bc1.reshape(1, H),
      wc2, bc2.reshape(1, wc2.shape[1]))


def kernel(x, edge_index, batch, w1a, b1a, w2a, b2a, w1b, b1b, w2b, b2b,
           w1c, b1c, w2c, b2c, wa, ba, wp, bp, wc1, bc1, wc2, bc2):
    n = x.shape[0]
    src = edge_index[0]
    dst = edge_index[1]
    x_pad = jnp.pad(x, ((0, N_PAD - n), (0, 0)))
    batch2d = jnp.pad(batch, (0, N_PAD - n),
                      constant_values=NUM_SEG).reshape(N_PAD, 1)
    zeros = jnp.zeros((N_PAD, H), jnp.float32)

    y1 = _tc_matmul(x_pad, w1a)
    a1 = _sc_aggregate(y1, src, dst, zeros)
    y2 = _tc_mlp(y1, a1[0], a1[1], b1a, w2a, b2a, w1b)
    a2 = _sc_aggregate(y2, src, dst, zeros)
    y3 = _tc_mlp(y2, a2[0], a2[1], b1b, w2b, b2b, w1c)
    a3 = _sc_aggregate(y3, src, dst, zeros)
    logits, ge, h3, scores = _tc_final(
        y3, a3[0], a3[1], b1c, w2c, b2c, batch2d, wa, ba, wp, bp,
        wc1, bc1, wc2, bc2, n, wc2.shape[1])
    return (logits, ge, h3[:n], scores[:n])


# speedup vs baseline: 3.9298x; 3.9298x over previous
import functools

import jax
import jax.numpy as jnp
from jax import lax
from jax.experimental import pallas as pl
from jax.experimental.pallas import tpu as pltpu
from jax.experimental.pallas import tpu_sc as plsc

N_PAD = 10240
H = 64
NUM_SEG = 64
CHUNK = 128
NUM_WORKERS = 32
ROWS_PER_SUB = N_PAD // 16


def _sc_aggregate(y, src, dst, zeros):
    E = src.shape[0]
    D = y.shape[1]
    nchunk = E // CHUNK
    per_tile = nchunk // NUM_WORKERS
    extra = nchunk - per_tile * NUM_WORKERS
    iters = per_tile + (1 if extra else 0)
    mesh = plsc.VectorSubcoreMesh(core_axis_name="c", subcore_axis_name="s")

    @functools.partial(
        pl.kernel,
        mesh=mesh,
        out_type=jax.ShapeDtypeStruct((2, N_PAD, D), jnp.float32),
        scratch_types=[
            pltpu.VMEM((CHUNK,), jnp.int32),
            pltpu.VMEM((CHUNK,), jnp.int32),
            pltpu.VMEM((CHUNK, D), jnp.float32),
            pltpu.VMEM_SHARED((N_PAD, D), jnp.float32),
            pltpu.SemaphoreType.DMA,
        ],
        compiler_params=pltpu.CompilerParams(use_tc_tiling_on_sc=False),
    )
    def agg(y_hbm, src_hbm, dst_hbm, z_hbm, out_hbm, src_v, dst_v, rows_v,
            acc_sh, sem):
        c = lax.axis_index("c")
        s = lax.axis_index("s")
        wid = c * 16 + s
        my_start = wid * per_tile + jnp.minimum(wid, extra)
        my_count = per_tile + jnp.where(wid < extra, 1, 0)
        pltpu.sync_copy(z_hbm.at[pl.ds(s * ROWS_PER_SUB, ROWS_PER_SUB)],
                        acc_sh.at[pl.ds(s * ROWS_PER_SUB, ROWS_PER_SUB)])
        plsc.subcore_barrier()

        def body(j, carry):
            k = my_start + j

            @pl.when(j < my_count)
            def _():
                base = k * CHUNK
                pltpu.sync_copy(src_hbm.at[pl.ds(base, CHUNK)], src_v)
                pltpu.sync_copy(dst_hbm.at[pl.ds(base, CHUNK)], dst_v)
                pltpu.async_copy(y_hbm.at[src_v], rows_v, sem).wait()
                pltpu.sync_copy(rows_v, acc_sh.at[dst_v], add=True)

            return carry

        lax.fori_loop(0, iters, body, 0)
        plsc.subcore_barrier()
        pltpu.sync_copy(acc_sh.at[pl.ds(s * ROWS_PER_SUB, ROWS_PER_SUB)],
                        out_hbm.at[c, pl.ds(s * ROWS_PER_SUB, ROWS_PER_SUB)])

    return agg(y, src, dst, zeros)


def _tc_gin(x, a0, a1, w1, b1, w2, b2):
    BR = 1024
    K = x.shape[1]

    def body(x_ref, a0_ref, a1_ref, w1_ref, b1_ref, w2_ref, b2_ref, o_ref):
        t = x_ref[...] + (a0_ref[...] + a1_ref[...])
        u = jnp.maximum(jnp.dot(t, w1_ref[...],
                                preferred_element_type=jnp.float32)
                        + b1_ref[...], 0.0)
        o_ref[...] = jnp.maximum(jnp.dot(u, w2_ref[...],
                                         preferred_element_type=jnp.float32)
                                 + b2_ref[...], 0.0)

    row = pl.BlockSpec((BR, K), lambda i: (i, 0))
    return pl.pallas_call(
        body,
        grid=(N_PAD // BR,),
        in_specs=[
            row, row, row,
            pl.BlockSpec((K, H), lambda i: (0, 0)),
            pl.BlockSpec((1, H), lambda i: (0, 0)),
            pl.BlockSpec((H, H), lambda i: (0, 0)),
            pl.BlockSpec((1, H), lambda i: (0, 0)),
        ],
        out_specs=pl.BlockSpec((BR, H), lambda i: (i, 0)),
        out_shape=jax.ShapeDtypeStruct((N_PAD, H), jnp.float32),
    )(x, a0, a1, w1, b1.reshape(1, H), w2, b2.reshape(1, H))


def _tc_final(x, a0, a1, w1, b1, w2, b2, batch2d, wa, ba, wp, bp, wc1, bc1,
              wc2, bc2, n_valid, out_dim):
    def body(x_ref, a0_ref, a1_ref, w1_ref, b1_ref, w2_ref, b2_ref,
             batch_ref, wa_ref, ba_ref, wp_ref, bp_ref, wc1_ref, bc1_ref,
             wc2_ref, bc2_ref, logits_ref, ge_ref, h_ref, scores_ref):
        t = x_ref[...] + (a0_ref[...] + a1_ref[...])
        u = jnp.maximum(jnp.dot(t, w1_ref[...],
                                preferred_element_type=jnp.float32)
                        + b1_ref[...], 0.0)
        h = jnp.maximum(jnp.dot(u, w2_ref[...],
                                preferred_element_type=jnp.float32)
                        + b2_ref[...], 0.0)
        h_ref[...] = h

        valid = (lax.broadcasted_iota(jnp.int32, (N_PAD, 1), 0) < n_valid)
        seg_ids = lax.broadcasted_iota(jnp.int32, (1, NUM_SEG), 1)
        onehot = jnp.logical_and(batch_ref[...] == seg_ids, valid)
        onehot_f = onehot.astype(jnp.float32)
        neg_inf = jnp.float32(-jnp.inf)

        def seg_softmax(logits):
            lg = jnp.where(valid, logits, 0.0)
            m = jnp.max(jnp.where(onehot, lg, neg_inf), axis=0,
                        keepdims=True)
            m = jnp.where(jnp.isfinite(m), m, 0.0)
            m_node = jnp.sum(onehot_f * m, axis=1, keepdims=True)
            e = jnp.exp(lg - m_node)
            e = jnp.where(valid, e, 0.0)
            d = jnp.sum(onehot_f * e, axis=0, keepdims=True)
            d_node = jnp.sum(onehot_f * d, axis=1, keepdims=True)
            return e / (d_node + 1e-16)

        sa = jnp.dot(h, wa_ref[...], preferred_element_type=jnp.float32) \
            + ba_ref[...]
        sp = jnp.dot(h, wp_ref[...], preferred_element_type=jnp.float32) \
            + bp_ref[...]
        scores_ref[...] = seg_softmax(sa)
        attn = seg_softmax(sp)

        ge = lax.dot_general(onehot_f, attn * h, (((0,), (0,)), ((), ())),
                             preferred_element_type=jnp.float32)
        ge_ref[...] = ge
        hc = jnp.maximum(jnp.dot(ge, wc1_ref[...],
                                 preferred_element_type=jnp.float32)
                         + bc1_ref[...], 0.0)
        logits_ref[...] = jnp.dot(hc, wc2_ref[...],
                                  preferred_element_type=jnp.float32) \
            + bc2_ref[...]

    return pl.pallas_call(
        body,
        out_shape=(
            jax.ShapeDtypeStruct((NUM_SEG, out_dim), jnp.float32),
            jax.ShapeDtypeStruct((NUM_SEG, H), jnp.float32),
            jax.ShapeDtypeStruct((N_PAD, H), jnp.float32),
            jax.ShapeDtypeStruct((N_PAD, 1), jnp.float32),
        ),
        compiler_params=pltpu.CompilerParams(
            vmem_limit_bytes=100 * 1024 * 1024),
    )(x, a0, a1, w1, b1.reshape(1, H), w2, b2.reshape(1, H), batch2d,
      wa, ba.reshape(1, 1), wp, bp.reshape(1, 1), wc1, bc1.reshape(1, H),
      wc2, bc2.reshape(1, wc2.shape[1]))


def kernel(x, edge_index, batch, w1a, b1a, w2a, b2a, w1b, b1b, w2b, b2b,
           w1c, b1c, w2c, b2c, wa, ba, wp, bp, wc1, bc1, wc2, bc2):
    n = x.shape[0]
    d_in = x.shape[1]
    perm = jnp.argsort(edge_index[1], stable=True)
    src = edge_index[0][perm]
    dst = edge_index[1][perm]
    x_pad = jnp.pad(x, ((0, N_PAD - n), (0, 0)))
    batch2d = jnp.pad(batch, (0, N_PAD - n),
                      constant_values=NUM_SEG).reshape(N_PAD, 1)
    zeros_in = jnp.zeros((N_PAD, d_in), jnp.float32)
    zeros_h = jnp.zeros((N_PAD, H), jnp.float32)

    a = _sc_aggregate(x_pad, src, dst, zeros_in)
    h1 = _tc_gin(x_pad, a[0], a[1], w1a, b1a, w2a, b2a)
    a = _sc_aggregate(h1, src, dst, zeros_h)
    h2 = _tc_gin(h1, a[0], a[1], w1b, b1b, w2b, b2b)
    a = _sc_aggregate(h2, src, dst, zeros_h)
    logits, ge, h3, scores = _tc_final(
        h2, a[0], a[1], w1c, b1c, w2c, b2c, batch2d, wa, ba, wp, bp,
        wc1, bc1, wc2, bc2, n, wc2.shape[1])
    return (logits, ge, h3[:n], scores[:n])


# speedup vs baseline: 4.3532x; 1.1078x over previous
import functools

import jax
import jax.numpy as jnp
from jax import lax
from jax.experimental import pallas as pl
from jax.experimental.pallas import tpu as pltpu
from jax.experimental.pallas import tpu_sc as plsc

N_PAD = 10240
H = 64
NUM_SEG = 64
CHUNK = 128
NUM_WORKERS = 32
ROWS_PER_SUB = N_PAD // 16


def _sc_aggregate(y, ei, zeros):
    E = ei.shape[1]
    D = y.shape[1]
    nchunk = E // CHUNK
    per_tile = nchunk // NUM_WORKERS
    extra = nchunk - per_tile * NUM_WORKERS
    iters = per_tile + (1 if extra else 0)
    mesh = plsc.VectorSubcoreMesh(core_axis_name="c", subcore_axis_name="s")

    @functools.partial(
        pl.kernel,
        mesh=mesh,
        out_type=jax.ShapeDtypeStruct((2, N_PAD, D), jnp.float32),
        scratch_types=[
            pltpu.VMEM((2, CHUNK), jnp.int32),
            pltpu.VMEM((CHUNK, D), jnp.float32),
            pltpu.VMEM_SHARED((N_PAD, D), jnp.float32),
            pltpu.SemaphoreType.DMA,
        ],
        compiler_params=pltpu.CompilerParams(use_tc_tiling_on_sc=False),
    )
    def agg(ei_hbm, y_hbm, z_hbm, out_hbm, idx_v, rows_v, acc_sh, sem):
        c = lax.axis_index("c")
        s = lax.axis_index("s")
        wid = c * 16 + s
        my_start = wid * per_tile + jnp.minimum(wid, extra)
        my_count = per_tile + jnp.where(wid < extra, 1, 0)
        pltpu.sync_copy(z_hbm.at[pl.ds(s * ROWS_PER_SUB, ROWS_PER_SUB)],
                        acc_sh.at[pl.ds(s * ROWS_PER_SUB, ROWS_PER_SUB)])
        plsc.subcore_barrier()

        def body(j, carry):
            k = my_start + j

            @pl.when(j < my_count)
            def _():
                base = k * CHUNK
                pltpu.sync_copy(ei_hbm.at[:, pl.ds(base, CHUNK)], idx_v)
                pltpu.async_copy(y_hbm.at[idx_v.at[0]], rows_v, sem).wait()
                pltpu.sync_copy(rows_v, acc_sh.at[idx_v.at[1]], add=True)

            return carry

        lax.fori_loop(0, iters, body, 0)
        plsc.subcore_barrier()
        pltpu.sync_copy(acc_sh.at[pl.ds(s * ROWS_PER_SUB, ROWS_PER_SUB)],
                        out_hbm.at[c, pl.ds(s * ROWS_PER_SUB, ROWS_PER_SUB)])

    return agg(ei, y, zeros)


def _tc_gin(x, a0, a1, w1, b1, w2, b2):
    BR = 1024
    K = x.shape[1]

    def body(x_ref, a0_ref, a1_ref, w1_ref, b1_ref, w2_ref, b2_ref, o_ref):
        t = x_ref[...] + (a0_ref[...] + a1_ref[...])
        u = jnp.maximum(jnp.dot(t, w1_ref[...],
                                preferred_element_type=jnp.float32)
                        + b1_ref[...], 0.0)
        o_ref[...] = jnp.maximum(jnp.dot(u, w2_ref[...],
                                         preferred_element_type=jnp.float32)
                                 + b2_ref[...], 0.0)

    row = pl.BlockSpec((BR, K), lambda i: (i, 0))
    return pl.pallas_call(
        body,
        grid=(N_PAD // BR,),
        in_specs=[
            row, row, row,
            pl.BlockSpec((K, H), lambda i: (0, 0)),
            pl.BlockSpec((1, H), lambda i: (0, 0)),
            pl.BlockSpec((H, H), lambda i: (0, 0)),
            pl.BlockSpec((1, H), lambda i: (0, 0)),
        ],
        out_specs=pl.BlockSpec((BR, H), lambda i: (i, 0)),
        out_shape=jax.ShapeDtypeStruct((N_PAD, H), jnp.float32),
    )(x, a0, a1, w1, b1.reshape(1, H), w2, b2.reshape(1, H))


def _tc_final(x, a0, a1, w1, b1, w2, b2, batch2d, wa, ba, wp, bp, wc1, bc1,
              wc2, bc2, n_valid, out_dim):
    def body(x_ref, a0_ref, a1_ref, w1_ref, b1_ref, w2_ref, b2_ref,
             batch_ref, wa_ref, ba_ref, wp_ref, bp_ref, wc1_ref, bc1_ref,
             wc2_ref, bc2_ref, logits_ref, ge_ref, h_ref, scores_ref):
        t = x_ref[...] + (a0_ref[...] + a1_ref[...])
        u = jnp.maximum(jnp.dot(t, w1_ref[...],
                                preferred_element_type=jnp.float32)
                        + b1_ref[...], 0.0)
        h = jnp.maximum(jnp.dot(u, w2_ref[...],
                                preferred_element_type=jnp.float32)
                        + b2_ref[...], 0.0)
        h_ref[...] = h

        valid = (lax.broadcasted_iota(jnp.int32, (N_PAD, 1), 0) < n_valid)
        seg_ids = lax.broadcasted_iota(jnp.int32, (1, NUM_SEG), 1)
        onehot = jnp.logical_and(batch_ref[...] == seg_ids, valid)
        onehot_f = onehot.astype(jnp.float32)
        neg_inf = jnp.float32(-jnp.inf)

        def seg_softmax(logits):
            lg = jnp.where(valid, logits, 0.0)
            m = jnp.max(jnp.where(onehot, lg, neg_inf), axis=0,
                        keepdims=True)
            m = jnp.where(jnp.isfinite(m), m, 0.0)
            m_node = jnp.sum(onehot_f * m, axis=1, keepdims=True)
            e = jnp.exp(lg - m_node)
            e = jnp.where(valid, e, 0.0)
            d = jnp.sum(onehot_f * e, axis=0, keepdims=True)
            d_node = jnp.sum(onehot_f * d, axis=1, keepdims=True)
            return e / (d_node + 1e-16)

        sa = jnp.dot(h, wa_ref[...], preferred_element_type=jnp.float32) \
            + ba_ref[...]
        sp = jnp.dot(h, wp_ref[...], preferred_element_type=jnp.float32) \
            + bp_ref[...]
        scores_ref[...] = seg_softmax(sa)
        attn = seg_softmax(sp)

        ge = lax.dot_general(onehot_f, attn * h, (((0,), (0,)), ((), ())),
                             preferred_element_type=jnp.float32)
        ge_ref[...] = ge
        hc = jnp.maximum(jnp.dot(ge, wc1_ref[...],
                                 preferred_element_type=jnp.float32)
                         + bc1_ref[...], 0.0)
        logits_ref[...] = jnp.dot(hc, wc2_ref[...],
                                  preferred_element_type=jnp.float32) \
            + bc2_ref[...]

    return pl.pallas_call(
        body,
        out_shape=(
            jax.ShapeDtypeStruct((NUM_SEG, out_dim), jnp.float32),
            jax.ShapeDtypeStruct((NUM_SEG, H), jnp.float32),
            jax.ShapeDtypeStruct((N_PAD, H), jnp.float32),
            jax.ShapeDtypeStruct((N_PAD, 1), jnp.float32),
        ),
        compiler_params=pltpu.CompilerParams(
            vmem_limit_bytes=100 * 1024 * 1024),
    )(x, a0, a1, w1, b1.reshape(1, H), w2, b2.reshape(1, H), batch2d,
      wa, ba.reshape(1, 1), wp, bp.reshape(1, 1), wc1, bc1.reshape(1, H),
      wc2, bc2.reshape(1, wc2.shape[1]))


def kernel(x, edge_index, batch, w1a, b1a, w2a, b2a, w1b, b1b, w2b, b2b,
           w1c, b1c, w2c, b2c, wa, ba, wp, bp, wc1, bc1, wc2, bc2):
    n = x.shape[0]
    d_in = x.shape[1]
    perm = jnp.argsort(edge_index[1], stable=True)
    ei_sorted = edge_index[:, perm]
    x_pad = jnp.pad(x, ((0, N_PAD - n), (0, 0)))
    batch2d = jnp.pad(batch, (0, N_PAD - n),
                      constant_values=NUM_SEG).reshape(N_PAD, 1)
    zeros_in = jnp.zeros((N_PAD, d_in), jnp.float32)
    zeros_h = jnp.zeros((N_PAD, H), jnp.float32)

    a = _sc_aggregate(x_pad, ei_sorted, zeros_in)
    h1 = _tc_gin(x_pad, a[0], a[1], w1a, b1a, w2a, b2a)
    a = _sc_aggregate(h1, ei_sorted, zeros_h)
    h2 = _tc_gin(h1, a[0], a[1], w1b, b1b, w2b, b2b)
    a = _sc_aggregate(h2, ei_sorted, zeros_h)
    logits, ge, h3, scores = _tc_final(
        h2, a[0], a[1], w1c, b1c, w2c, b2c, batch2d, wa, ba, wp, bp,
        wc1, bc1, wc2, bc2, n, wc2.shape[1])
    return (logits, ge, h3[:n], scores[:n])
